# baseline (speedup 1.0000x reference)
import functools

import jax
import jax.numpy as jnp
from jax import lax
from jax.experimental import pallas as pl
from jax.experimental.pallas import tpu as pltpu
from jax.experimental.pallas import tpu_sc as plsc

NC = 2
NS = 16
NW = NC * NS
CHUNK = 128


NBUF = 2
GRP = 8


def _sc_segment_sum(x, src2d, dst2d, zeros, n_acc, chunks_per_tile):
    N, F = x.shape
    rows_init = n_acc // NS
    rows_out = rows_init
    ngroups = chunks_per_tile // GRP
    mesh = plsc.VectorSubcoreMesh(core_axis_name="c", subcore_axis_name="s")

    @functools.partial(
        pl.kernel,
        mesh=mesh,
        out_type=jax.ShapeDtypeStruct((NC, n_acc, F), jnp.float32),
        scratch_types=[pltpu.VMEM((GRP, CHUNK), jnp.int32) for _ in range(4)]
        + [pltpu.VMEM((CHUNK, F), jnp.float32) for _ in range(NBUF)]
        + [
            pltpu.VMEM_SHARED((n_acc, F), jnp.float32),
            pltpu.SemaphoreType.DMA,
            pltpu.SemaphoreType.DMA,
        ],
    )
    def sc_kernel(x_hbm, src_hbm, dst_hbm, zeros_hbm, out_hbm,
                  ssrc0, ssrc1, sdst0, sdst1, *rest):
        ssrc = (ssrc0, ssrc1)
        sdst = (sdst0, sdst1)
        rbufs = rest[:NBUF]
        acc, gsem, ssem = rest[NBUF], rest[NBUF + 1], rest[NBUF + 2]
        c = lax.axis_index("c")
        s = lax.axis_index("s")
        wid = c * NS + s
        chunk0 = wid * chunks_per_tile

        def fire_slab(g):
            pltpu.async_copy(src_hbm.at[pl.ds(chunk0 + g * GRP, GRP)],
                             ssrc[g % 2], ssem)
            pltpu.async_copy(dst_hbm.at[pl.ds(chunk0 + g * GRP, GRP)],
                             sdst[g % 2], ssem)

        def drain_slab(g):
            pltpu.make_async_copy(src_hbm.at[pl.ds(0, GRP)], ssrc[g % 2],
                                  ssem).wait()
            pltpu.make_async_copy(dst_hbm.at[pl.ds(0, GRP)], sdst[g % 2],
                                  ssem).wait()

        def fire_g(ch):
            g, k = ch // GRP, ch % GRP
            pltpu.async_copy(x_hbm.at[ssrc[g % 2].at[k]], rbufs[ch % NBUF],
                             gsem)

        def drain_g(ch):
            pltpu.make_async_copy(x_hbm.at[pl.ds(0, CHUNK)],
                                  rbufs[ch % NBUF], gsem).wait()

        def scatter(ch):
            g, k = ch // GRP, ch % GRP
            pltpu.sync_copy(rbufs[ch % NBUF], acc.at[sdst[g % 2].at[k]],
                            add=True)

        fire_slab(0)
        if ngroups > 1:
            fire_slab(1)
        pltpu.sync_copy(zeros_hbm, acc.at[pl.ds(s * rows_init, rows_init)])
        drain_slab(0)
        plsc.subcore_barrier()

        nch = chunks_per_tile
        for ch in range(min(NBUF, nch)):
            fire_g(ch)
        for ch in range(nch):
            drain_g(ch)
            scatter(ch)
            nx = ch + NBUF
            if nx < nch:
                gn = nx // GRP
                if nx % GRP == 0:
                    drain_slab(gn)
                fire_g(nx)
                if nx % GRP == 1 and gn + 1 < ngroups:
                    fire_slab(gn + 1)

        plsc.subcore_barrier()
        pltpu.sync_copy(
            acc.at[pl.ds(s * rows_out, rows_out)],
            out_hbm.at[c, pl.ds(s * rows_out, rows_out)],
        )

    return sc_kernel(x, src2d, dst2d, zeros)


def _tc_finish(partials, W, b, N):
    _, _, F = partials.shape
    D = W.shape[1]
    blk = 1000

    def body(p_ref, w_ref, b_ref, o_ref):
        s = p_ref[0] + p_ref[1]
        nrm = jnp.sqrt(jnp.sum(s * s, axis=1, keepdims=True))
        s = s / jnp.maximum(nrm, 1e-12)
        o_ref[...] = (
            jnp.dot(s, w_ref[...], preferred_element_type=jnp.float32)
            + b_ref[...]
        )

    return pl.pallas_call(
        body,
        grid=(N // blk,),
        in_specs=[
            pl.BlockSpec((2, blk, F), lambda i: (0, i, 0)),
            pl.BlockSpec((F, D), lambda i: (0, 0)),
            pl.BlockSpec((1, D), lambda i: (0, 0)),
        ],
        out_specs=pl.BlockSpec((blk, D), lambda i: (i, 0)),
        out_shape=jax.ShapeDtypeStruct((N, D), jnp.float32),
    )(partials, W, b.reshape(1, D))


def kernel(input_features, edges, W, b):
    N, F = input_features.shape
    E = edges.shape[0]
    e32 = edges.astype(jnp.int32)
    chunks_per_tile = -(-E // (NW * CHUNK * GRP)) * GRP
    e_pad = NW * CHUNK * chunks_per_tile
    pad = e_pad - E
    src = jnp.concatenate([e32[:, 1], jnp.zeros((pad,), jnp.int32)])
    dst = jnp.concatenate([e32[:, 0], jnp.full((pad,), N, jnp.int32)])
    src = src.reshape(e_pad // CHUNK, CHUNK)
    dst = dst.reshape(e_pad // CHUNK, CHUNK)
    n_acc = ((N + 1 + NS * 8 - 1) // (NS * 8)) * (NS * 8)
    zeros = jnp.zeros((n_acc // NS, F), jnp.float32)
    partials = _sc_segment_sum(input_features, src, dst, zeros,
                               n_acc, chunks_per_tile)
    return _tc_finish(partials, W, b, N)


# speedup vs baseline: 1.0047x; 1.0047x over previous
import functools

import jax
import jax.numpy as jnp
from jax import lax
from jax.experimental import pallas as pl
from jax.experimental.pallas import tpu as pltpu
from jax.experimental.pallas import tpu_sc as plsc

NC = 2
NS = 16
NW = NC * NS
CHUNK = 128


def _sc_segment_sum(x, src2d, dst2d, zeros, n_acc, chunks_per_tile):
    N, F = x.shape
    rows_init = n_acc // NS
    rows_out = rows_init
    nch = chunks_per_tile
    mesh = plsc.VectorSubcoreMesh(core_axis_name="c", subcore_axis_name="s")

    @functools.partial(
        pl.kernel,
        mesh=mesh,
        out_type=jax.ShapeDtypeStruct((NC, n_acc, F), jnp.float32),
        scratch_types=[pltpu.VMEM((1, CHUNK), jnp.int32) for _ in range(8)]
        + [pltpu.VMEM((CHUNK, F), jnp.float32) for _ in range(2)]
        + [
            pltpu.VMEM_SHARED((n_acc, F), jnp.float32),
            pltpu.SemaphoreType.DMA,
            pltpu.SemaphoreType.DMA,
        ],
    )
    def sc_kernel(x_hbm, src_hbm, dst_hbm, zeros_hbm, out_hbm, *rest):
        isrc = rest[0:4]
        idst = rest[4:8]
        rbufs = rest[8:10]
        acc, gsem, isem = rest[10], rest[11], rest[12]
        c = lax.axis_index("c")
        s = lax.axis_index("s")
        wid = c * NS + s
        chunk0 = wid * nch

        def fire_idx(ch, q):
            pltpu.async_copy(src_hbm.at[pl.ds(chunk0 + ch, 1)], isrc[q], isem)
            pltpu.async_copy(dst_hbm.at[pl.ds(chunk0 + ch, 1)], idst[q], isem)

        def drain_idx(q):
            pltpu.make_async_copy(src_hbm.at[pl.ds(0, 1)], isrc[q],
                                  isem).wait()
            pltpu.make_async_copy(dst_hbm.at[pl.ds(0, 1)], idst[q],
                                  isem).wait()

        def fire_g(q, p):
            pltpu.async_copy(x_hbm.at[isrc[q].at[0]], rbufs[p], gsem)

        def drain_g(p):
            pltpu.make_async_copy(x_hbm.at[pl.ds(0, CHUNK)], rbufs[p],
                                  gsem).wait()

        def scatter(q, p):
            pltpu.sync_copy(rbufs[p], acc.at[idst[q].at[0]], add=True)

        fire_idx(0, 0)
        fire_idx(1, 1)
        fire_idx(2, 2)
        pltpu.sync_copy(zeros_hbm, acc.at[pl.ds(s * rows_init, rows_init)])
        drain_idx(0)
        fire_g(0, 0)
        plsc.subcore_barrier()

        def step(ch, k, fire_gn, fire_ix):
            if fire_gn:
                drain_idx((k + 1) % 4)
                fire_g((k + 1) % 4, (k + 1) % 2)
            if fire_ix:
                fire_idx(ch + 3, (k + 3) % 4)
            drain_g(k % 2)
            scatter(k % 4, k % 2)

        def body(i, carry):
            ch = i * 4
            for k in range(4):
                step(ch + k, k, True, True)
            return carry

        lax.fori_loop(0, nch // 4 - 1, body, 0)
        base = nch - 4
        step(base, 0, True, True)
        step(base + 1, 1, True, False)
        step(base + 2, 2, True, False)
        step(base + 3, 3, False, False)

        plsc.subcore_barrier()
        pltpu.sync_copy(
            acc.at[pl.ds(s * rows_out, rows_out)],
            out_hbm.at[c, pl.ds(s * rows_out, rows_out)],
        )

    return sc_kernel(x, src2d, dst2d, zeros)


def _tc_finish(partials, W, b, N):
    _, _, F = partials.shape
    D = W.shape[1]
    blk = 1000

    def body(p_ref, w_ref, b_ref, o_ref):
        s = p_ref[0] + p_ref[1]
        nrm = jnp.sqrt(jnp.sum(s * s, axis=1, keepdims=True))
        s = s / jnp.maximum(nrm, 1e-12)
        o_ref[...] = (
            jnp.dot(s, w_ref[...], preferred_element_type=jnp.float32)
            + b_ref[...]
        )

    return pl.pallas_call(
        body,
        grid=(N // blk,),
        in_specs=[
            pl.BlockSpec((2, blk, F), lambda i: (0, i, 0)),
            pl.BlockSpec((F, D), lambda i: (0, 0)),
            pl.BlockSpec((1, D), lambda i: (0, 0)),
        ],
        out_specs=pl.BlockSpec((blk, D), lambda i: (i, 0)),
        out_shape=jax.ShapeDtypeStruct((N, D), jnp.float32),
    )(partials, W, b.reshape(1, D))


def kernel(input_features, edges, W, b):
    N, F = input_features.shape
    E = edges.shape[0]
    e32 = edges.astype(jnp.int32)
    chunks_per_tile = -(-E // (NW * CHUNK * 4)) * 4
    e_pad = NW * CHUNK * chunks_per_tile
    pad = e_pad - E
    src = jnp.concatenate([e32[:, 1], jnp.zeros((pad,), jnp.int32)])
    dst = jnp.concatenate([e32[:, 0], jnp.full((pad,), N, jnp.int32)])
    src = src.reshape(e_pad // CHUNK, CHUNK)
    dst = dst.reshape(e_pad // CHUNK, CHUNK)
    n_acc = ((N + 1 + NS * 8 - 1) // (NS * 8)) * (NS * 8)
    zeros = jnp.zeros((n_acc // NS, F), jnp.float32)
    partials = _sc_segment_sum(input_features, src, dst, zeros,
                               n_acc, chunks_per_tile)
    return _tc_finish(partials, W, b, N)
